# baseline (speedup 1.0000x reference)
import functools

import jax
import jax.numpy as jnp
from jax import lax
from jax.experimental import pallas as pl
from jax.experimental.pallas import tpu as pltpu
from jax.experimental.pallas import tpu_sc as plsc

NUM_CLASS = 100000
BATCH = 4096
CTX_DIM = 512
N_CLS_CTX = 4
SEQ_LEN = 77
PRE = 5
SUF = SEQ_LEN - PRE - N_CLS_CTX
ROW = N_CLS_CTX * CTX_DIM

_INFO = plsc.get_sparse_core_info()
_NC, _NS = _INFO.num_cores, _INFO.num_subcores
_NW = _NC * _NS
B_PER_W = BATCH // _NW
CHUNK = 32
N_CHUNKS = B_PER_W // CHUNK


def _sc_gather(label, cls2d):
    mesh = plsc.VectorSubcoreMesh(core_axis_name="c", subcore_axis_name="s")

    @functools.partial(
        pl.kernel,
        mesh=mesh,
        out_type=jax.ShapeDtypeStruct((BATCH, ROW), jnp.float32),
        scratch_types=[
            pltpu.VMEM((CHUNK,), jnp.int32),
            pltpu.VMEM((CHUNK, ROW), jnp.float32),
            pltpu.SemaphoreType.DMA,
        ],
    )
    def k(cls_hbm, label_hbm, out_hbm, idx_v, rows_v, sem):
        wid = lax.axis_index("s") * _NC + lax.axis_index("c")
        base = wid * B_PER_W
        for c in range(N_CHUNKS):
            off = base + c * CHUNK
            pltpu.sync_copy(label_hbm.at[pl.ds(off, CHUNK)], idx_v)
            pltpu.async_copy(cls_hbm.at[idx_v], rows_v, sem).wait()
            pltpu.sync_copy(rows_v, out_hbm.at[pl.ds(off, CHUNK)])

    return k(cls2d, label)


BBLK = 64


def _tc_assemble(cls_g, prefix, suffix):
    def body(cls_ref, pre_ref, suf_ref, out_ref):
        out_ref[:, 0:PRE, :] = jnp.broadcast_to(
            pre_ref[...], (BBLK, PRE, CTX_DIM))
        out_ref[:, PRE:PRE + N_CLS_CTX, :] = cls_ref[...]
        out_ref[:, PRE + N_CLS_CTX:, :] = jnp.broadcast_to(
            suf_ref[...], (BBLK, SUF, CTX_DIM))

    return pl.pallas_call(
        body,
        grid=(BATCH // BBLK,),
        in_specs=[
            pl.BlockSpec((BBLK, N_CLS_CTX, CTX_DIM), lambda i: (i, 0, 0)),
            pl.BlockSpec((1, PRE, CTX_DIM), lambda i: (0, 0, 0)),
            pl.BlockSpec((1, SUF, CTX_DIM), lambda i: (0, 0, 0)),
        ],
        out_specs=pl.BlockSpec((BBLK, SEQ_LEN, CTX_DIM), lambda i: (i, 0, 0)),
        out_shape=jax.ShapeDtypeStruct((BATCH, SEQ_LEN, CTX_DIM), jnp.float32),
    )(cls_g, prefix, suffix)


def kernel(label, cls_ctx, token_prefix, token_suffix):
    cls2d = cls_ctx.reshape(NUM_CLASS, ROW)
    cls_g = _sc_gather(label.astype(jnp.int32), cls2d)
    cls_g3 = cls_g.reshape(BATCH, N_CLS_CTX, CTX_DIM)
    return _tc_assemble(cls_g3, token_prefix, token_suffix)
